# Initial kernel scaffold; baseline (speedup 1.0000x reference)
import jax
import jax.numpy as jnp
from jax.experimental import pallas as pl


def kernel(norm, pos, batch, params):
    raise NotImplementedError("write your pallas kernel here")


# speedup vs baseline: 135.8317x; 135.8317x over previous
import functools
import numpy as np
import jax
import jax.numpy as jnp
from jax.experimental import pallas as pl
from jax.experimental.pallas import tpu as pltpu

_B = 8
_P = 1024
_OUT = 40
_K = 64
_BN_EPS = 1e-05
_DEN = np.float32(np.sqrt(1.0 + _BN_EPS))


def _fps_body(M, pos_t_ref, q_ref):
    Bb = pos_t_ref.shape[0]
    P = pos_t_ref.shape[2]
    px = pos_t_ref[:, 0, :]
    py = pos_t_ref[:, 1, :]
    pz = pos_t_ref[:, 2, :]
    iota = jax.lax.broadcasted_iota(jnp.int32, (Bb, P), 1)

    q_ref[0:1, :, :] = jnp.concatenate(
        [px[:, 0:1], py[:, 0:1], pz[:, 0:1]], axis=1)[None]
    dx = px - px[:, 0:1]
    dy = py - py[:, 0:1]
    dz = pz - pz[:, 0:1]
    mind0 = dx * dx + dy * dy + dz * dz

    def body(i, mind):
        mx = jnp.max(mind, axis=1, keepdims=True)
        cand = jnp.where(mind == mx, iota, P)
        nxt = jnp.min(cand, axis=1, keepdims=True)
        oh = iota == nxt
        nx = jnp.sum(jnp.where(oh, px, 0.0), axis=1, keepdims=True)
        ny = jnp.sum(jnp.where(oh, py, 0.0), axis=1, keepdims=True)
        nz = jnp.sum(jnp.where(oh, pz, 0.0), axis=1, keepdims=True)
        q_ref[pl.ds(i, 1), :, :] = jnp.concatenate([nx, ny, nz], axis=1)[None]
        ddx = px - nx
        ddy = py - ny
        ddz = pz - nz
        d = ddx * ddx + ddy * ddy + ddz * ddz
        return jnp.minimum(mind, d)

    jax.lax.fori_loop(1, M, body, mind0)


def _fps_call(pos_t, M):
    Bb, _, P = pos_t.shape
    out = pl.pallas_call(
        functools.partial(_fps_body, M),
        out_shape=jax.ShapeDtypeStruct((M, Bb, 3), jnp.float32),
    )(pos_t)
    return out


def _sa_body(P, Tq, Cx, Cout, r2,
             pos_t_ref, pos_r_ref, x_ref, q_ref,
             w1x_ref, w1p_ref, b1_ref, g1_ref, be1_ref,
             w2_ref, b2_ref, g2_ref, be2_ref,
             w3_ref, b3_ref, g3_ref, be3_ref,
             out_ref):
    q = q_ref[0]
    d2 = None
    for c in range(3):
        pc = pos_t_ref[0, c:c + 1, :]
        dc = q[:, c:c + 1] - pc
        d2 = dc * dc if d2 is None else d2 + dc * dc
    mask = d2 <= r2
    mi = mask.astype(jnp.int32)
    colio = jax.lax.broadcasted_iota(jnp.int32, (Tq, P), 1)
    cum = mi
    s = 1
    while s < P:
        sh = pltpu.roll(cum, s, 1)
        cum = cum + jnp.where(colio >= s, sh, 0)
        s *= 2
    rank = cum - mi
    count = cum[:, P - 1:P]

    kio3 = jax.lax.broadcasted_iota(jnp.int32, (Tq, _K, P), 1)
    S = jnp.logical_and(rank[:, None, :] == kio3, mask[:, None, :])
    S = S.astype(jnp.float32).reshape(Tq * _K, P)

    x_nb = jnp.dot(S, x_ref[0], preferred_element_type=jnp.float32)
    p_nb = jnp.dot(S, pos_r_ref[0], preferred_element_type=jnp.float32)
    qb = jnp.broadcast_to(q[:, None, :], (Tq, _K, 3)).reshape(Tq * _K, 3)
    relp = p_nb - qb

    h = (jnp.dot(x_nb, w1x_ref[...], preferred_element_type=jnp.float32)
         + jnp.dot(relp, w1p_ref[...], preferred_element_type=jnp.float32)
         + b1_ref[...])
    h = jax.nn.relu(h)
    h = g1_ref[...] * h / _DEN + be1_ref[...]
    h = jnp.dot(h, w2_ref[...], preferred_element_type=jnp.float32) + b2_ref[...]
    h = jax.nn.relu(h)
    h = g2_ref[...] * h / _DEN + be2_ref[...]
    h = jnp.dot(h, w3_ref[...], preferred_element_type=jnp.float32) + b3_ref[...]
    h = jax.nn.relu(h)
    h = g3_ref[...] * h / _DEN + be3_ref[...]

    h3 = h.reshape(Tq, _K, Cout)
    kio3d = jax.lax.broadcasted_iota(jnp.int32, (Tq, _K, Cout), 1)
    h3 = jnp.where(kio3d < count[:, :, None], h3, -jnp.inf)
    out_ref[0] = jnp.max(h3, axis=1)


def _sa_call(pos_t, pos_r, x, q, r, params, prefix, Tq=16):
    Bb, _, P = pos_t.shape
    M = q.shape[1]
    Cx = x.shape[2]
    w1 = params[prefix + '_W0']
    Cout = params[prefix + '_W2'].shape[1]
    r2 = np.float32(r * r)
    row = lambda v: v[None, :]
    args = (pos_t, pos_r, x, q,
            w1[:Cx], w1[Cx:], row(params[prefix + '_b0']),
            row(params[prefix + '_g0']), row(params[prefix + '_beta0']),
            params[prefix + '_W1'], row(params[prefix + '_b1']),
            row(params[prefix + '_g1']), row(params[prefix + '_beta1']),
            params[prefix + '_W2'], row(params[prefix + '_b2']),
            row(params[prefix + '_g2']), row(params[prefix + '_beta2']))
    full = lambda a: pl.BlockSpec(a.shape, lambda b, t: (0,) * a.ndim)
    cloud = lambda a: pl.BlockSpec((1,) + a.shape[1:], lambda b, t: (b,) + (0,) * (a.ndim - 1))
    in_specs = [cloud(pos_t), cloud(pos_r), cloud(x),
                pl.BlockSpec((1, Tq, 3), lambda b, t: (b, t, 0))]
    in_specs += [full(a) for a in args[4:]]
    out = pl.pallas_call(
        functools.partial(_sa_body, P, Tq, Cx, Cout, r2),
        grid=(Bb, M // Tq),
        in_specs=in_specs,
        out_specs=pl.BlockSpec((1, Tq, Cout), lambda b, t: (b, t, 0)),
        out_shape=jax.ShapeDtypeStruct((Bb, M, Cout), jnp.float32),
    )(*args)
    return out


def _head_body(Bb, M,
               x_ref, p_ref,
               w1x_ref, w1p_ref, b1_ref, g1_ref, be1_ref,
               w2_ref, b2_ref, g2_ref, be2_ref,
               w3_ref, b3_ref, g3_ref, be3_ref,
               l1w_ref, l1b_ref, l2w_ref, l2b_ref, l3w_ref, l3b_ref,
               out_ref):
    h = (jnp.dot(x_ref[...], w1x_ref[...], preferred_element_type=jnp.float32)
         + jnp.dot(p_ref[...], w1p_ref[...], preferred_element_type=jnp.float32)
         + b1_ref[...])
    h = jax.nn.relu(h)
    h = g1_ref[...] * h / _DEN + be1_ref[...]
    h = jnp.dot(h, w2_ref[...], preferred_element_type=jnp.float32) + b2_ref[...]
    h = jax.nn.relu(h)
    h = g2_ref[...] * h / _DEN + be2_ref[...]
    h = jnp.dot(h, w3_ref[...], preferred_element_type=jnp.float32) + b3_ref[...]
    h = jax.nn.relu(h)
    h = g3_ref[...] * h / _DEN + be3_ref[...]

    pooled = [jnp.max(h[b * M:(b + 1) * M, :], axis=0, keepdims=True)
              for b in range(Bb)]
    x = jnp.concatenate(pooled, axis=0)
    x = jax.nn.relu(jnp.dot(x, l1w_ref[...], preferred_element_type=jnp.float32)
                    + l1b_ref[...])
    x = jax.nn.relu(jnp.dot(x, l2w_ref[...], preferred_element_type=jnp.float32)
                    + l2b_ref[...])
    x = jnp.dot(x, l3w_ref[...], preferred_element_type=jnp.float32) + l3b_ref[...]
    mx = jnp.max(x, axis=1, keepdims=True)
    s = x - mx
    out_ref[...] = s - jnp.log(jnp.sum(jnp.exp(s), axis=1, keepdims=True))


def _head_call(x2, p2, params):
    Bb, M, C = x2.shape
    w1 = params['sa3_W0']
    row = lambda v: v[None, :]
    args = (x2.reshape(Bb * M, C), p2.reshape(Bb * M, 3),
            w1[:C], w1[C:], row(params['sa3_b0']),
            row(params['sa3_g0']), row(params['sa3_beta0']),
            params['sa3_W1'], row(params['sa3_b1']),
            row(params['sa3_g1']), row(params['sa3_beta1']),
            params['sa3_W2'], row(params['sa3_b2']),
            row(params['sa3_g2']), row(params['sa3_beta2']),
            params['lin1_W'], row(params['lin1_b']),
            params['lin2_W'], row(params['lin2_b']),
            params['lin3_W'], row(params['lin3_b']))
    out = pl.pallas_call(
        functools.partial(_head_body, Bb, M),
        out_shape=jax.ShapeDtypeStruct((Bb, _OUT), jnp.float32),
    )(*args)
    return out


def kernel(norm, pos, batch, params):
    del batch
    pos_b = pos.reshape(_B, _P, 3)
    x_b = norm.reshape(_B, _P, 3)
    pos_t = jnp.transpose(pos_b, (0, 2, 1))

    q1 = jnp.transpose(_fps_call(pos_t, _P // 2), (1, 0, 2))
    x1 = _sa_call(pos_t, pos_b, x_b, q1, 0.2, params, 'sa1')

    q1_t = jnp.transpose(q1, (0, 2, 1))
    q2 = jnp.transpose(_fps_call(q1_t, _P // 8), (1, 0, 2))
    x2 = _sa_call(q1_t, q1, x1, q2, 0.4, params, 'sa2')

    return _head_call(x2, q2, params)
